# baseline (speedup 1.0000x reference)
import functools
import math

import jax
import jax.numpy as jnp
from jax.experimental import pallas as pl
from jax.experimental.pallas import tpu as pltpu
from jax.experimental.pallas import tpu_sc as plsc

_TEMP = 0.05
_INV_TEMP = 1.0 / _TEMP
_A = _INV_TEMP * math.log2(math.e)
_B = 256
_D = 256
_N = 100000
_ROWS = 5000
_STEPS = _N // _ROWS


def _sc_gather_kernel(frgb_hbm, fnir_hbm, idx_hbm, grgb_hbm, gnir_hbm,
                      idx_v, rows_a, rows_b, sem_a, sem_b):
    nc = plsc.get_sparse_core_info().num_cores
    wid = jax.lax.axis_index("s") * nc + jax.lax.axis_index("c")
    base = wid * 8
    pltpu.sync_copy(idx_hbm.at[pl.ds(base, 8)], idx_v)
    cpa = pltpu.async_copy(frgb_hbm.at[idx_v], rows_a, sem_a)
    cpb = pltpu.async_copy(fnir_hbm.at[idx_v], rows_b, sem_b)
    cpa.wait()
    pltpu.sync_copy(rows_a, grgb_hbm.at[pl.ds(base, 8)])
    cpb.wait()
    pltpu.sync_copy(rows_b, gnir_hbm.at[pl.ds(base, 8)])


def _fused_loss_kernel(ir_ref, inr_ref, frgb_ref, fnir_ref, grgb_ref,
                       gnir_ref, out_rgb_ref, out_nir_ref,
                       irn, inrn, irs, inrs, se_rgb, se_nir):
    i = pl.program_id(0)

    @pl.when(i == 0)
    def _init():
        for src, dst, dsts in ((ir_ref, irn, irs), (inr_ref, inrn, inrs)):
            x = src[...]
            norm = jnp.sqrt(jnp.sum(x * x, axis=1, keepdims=True))
            xn = x / jnp.maximum(norm, 1e-12)
            dst[...] = xn
            dsts[...] = xn * _A
        zero = jnp.zeros((1, _B), jnp.float32)
        se_rgb[...] = zero
        se_nir[...] = zero

    for q, f_ref, se in ((irs, frgb_ref, se_rgb), (inrs, fnir_ref, se_nir)):
        d = jax.lax.dot_general(
            f_ref[...], q[...], (((1,), (1,)), ((), ())),
            precision=jax.lax.Precision.DEFAULT,
            preferred_element_type=jnp.float32)
        e = jnp.exp2(d.astype(jnp.bfloat16)).astype(jnp.float32)
        se[...] += jnp.sum(e, axis=0, keepdims=True)

    @pl.when(i == _STEPS - 1)
    def _finish():
        for q, g_ref, se, out in ((irn, grgb_ref, se_rgb, out_rgb_ref),
                                  (inrn, gnir_ref, se_nir, out_nir_ref)):
            tl_total = _INV_TEMP * jnp.sum(q[...] * g_ref[...])
            lse_total = jnp.sum(jnp.log(se[...]))
            out[...] = ((lse_total - tl_total) / _B).reshape(1, 1)


@functools.partial(jax.jit, static_argnames=())
def kernel(inputs_rgb, inputs_nir, targets, features_rgb, features_nir):
    g_rgb, g_nir = pl.kernel(
        _sc_gather_kernel,
        out_type=[jax.ShapeDtypeStruct((_B, _D), jnp.float32),
                  jax.ShapeDtypeStruct((_B, _D), jnp.float32)],
        mesh=plsc.VectorSubcoreMesh(core_axis_name="c", subcore_axis_name="s"),
        scratch_types=[
            pltpu.VMEM((8,), jnp.int32),
            pltpu.VMEM((8, _D), jnp.float32),
            pltpu.VMEM((8, _D), jnp.float32),
            pltpu.SemaphoreType.DMA,
            pltpu.SemaphoreType.DMA,
        ],
    )(features_rgb, features_nir, targets)

    full = lambda shape: pl.BlockSpec(shape, lambda i: (0, 0))
    out_rgb, out_nir = pl.pallas_call(
        _fused_loss_kernel,
        grid=(_STEPS,),
        in_specs=[
            full((_B, _D)),
            full((_B, _D)),
            pl.BlockSpec((_ROWS, _D), lambda i: (i, 0)),
            pl.BlockSpec((_ROWS, _D), lambda i: (i, 0)),
            full((_B, _D)),
            full((_B, _D)),
        ],
        out_specs=[full((1, 1)), full((1, 1))],
        out_shape=[jax.ShapeDtypeStruct((1, 1), jnp.float32),
                   jax.ShapeDtypeStruct((1, 1), jnp.float32)],
        scratch_shapes=[
            pltpu.VMEM((_B, _D), jnp.float32),
            pltpu.VMEM((_B, _D), jnp.float32),
            pltpu.VMEM((_B, _D), jnp.float32),
            pltpu.VMEM((_B, _D), jnp.float32),
            pltpu.VMEM((1, _B), jnp.float32),
            pltpu.VMEM((1, _B), jnp.float32),
        ],
        compiler_params=pltpu.CompilerParams(
            dimension_semantics=("arbitrary",)),
    )(inputs_rgb, inputs_nir, features_rgb, features_nir, g_rgb, g_nir)
    return (out_rgb.reshape(()), out_nir.reshape(()))


# speedup vs baseline: 1.2981x; 1.2981x over previous
import functools
import math

import jax
import jax.numpy as jnp
from jax.experimental import pallas as pl
from jax.experimental.pallas import tpu as pltpu

_TEMP = 0.05
_INV_TEMP = 1.0 / _TEMP
_A = _INV_TEMP * math.log2(math.e)
_B = 256
_D = 256
_N = 100000
_ROWS = 5000
_STEPS = _N // _ROWS
_GATHER_STEPS = 16
_PER_STEP = _B // _GATHER_STEPS
_DRAIN_LAG = 3


def _fused_loss_kernel(tgt_ref, ir_ref, inr_ref, frgb_ref, fnir_ref,
                       frgb_any, fnir_any,
                       out_rgb_ref, out_nir_ref,
                       irn, inrn, irs, inrs, se_rgb, se_nir, g_rgb, g_nir,
                       sem):
    i = pl.program_id(0)

    @pl.when(i == 0)
    def _init():
        for src, dst, dsts in ((ir_ref, irn, irs), (inr_ref, inrn, inrs)):
            x = src[...]
            norm = jnp.sqrt(jnp.sum(x * x, axis=1, keepdims=True))
            xn = x / jnp.maximum(norm, 1e-12)
            dst[...] = xn
            dsts[...] = xn * _A
        zero = jnp.zeros((1, _B), jnp.float32)
        se_rgb[...] = zero
        se_nir[...] = zero

    for q, f_ref, se in ((irs, frgb_ref, se_rgb), (inrs, fnir_ref, se_nir)):
        d = jax.lax.dot_general(
            f_ref[...], q[...], (((1,), (1,)), ((), ())),
            precision=jax.lax.Precision.DEFAULT,
            preferred_element_type=jnp.float32)
        e = jnp.exp2(d.astype(jnp.bfloat16)).astype(jnp.float32)
        se[...] += jnp.sum(e, axis=0, keepdims=True)

    @pl.when(i < _GATHER_STEPS)
    def _issue():
        def issue(j, _):
            t = tgt_ref[j]
            pltpu.make_async_copy(frgb_any.at[pl.ds(t, 1), :],
                                  g_rgb.at[pl.ds(j, 1), :], sem).start()
            pltpu.make_async_copy(fnir_any.at[pl.ds(t, 1), :],
                                  g_nir.at[pl.ds(j, 1), :], sem).start()
            return 0
        jax.lax.fori_loop(i * _PER_STEP, (i + 1) * _PER_STEP, issue, 0)

    @pl.when(jnp.logical_and(i >= _DRAIN_LAG, i < _DRAIN_LAG + _GATHER_STEPS))
    def _drain():
        def drain(j, _):
            pltpu.make_async_copy(frgb_any.at[pl.ds(0, 1), :],
                                  g_rgb.at[pl.ds(0, 1), :], sem).wait()
            pltpu.make_async_copy(fnir_any.at[pl.ds(0, 1), :],
                                  g_nir.at[pl.ds(0, 1), :], sem).wait()
            return 0
        jax.lax.fori_loop(0, _PER_STEP, drain, 0)

    @pl.when(i == _STEPS - 1)
    def _finish():
        for q, g, se, out in ((irn, g_rgb, se_rgb, out_rgb_ref),
                              (inrn, g_nir, se_nir, out_nir_ref)):
            tl_total = _INV_TEMP * jnp.sum(q[...] * g[...])
            lse_total = jnp.sum(jnp.log(se[...]))
            out[...] = ((lse_total - tl_total) / _B).reshape(1, 1)


@functools.partial(jax.jit, static_argnames=())
def kernel(inputs_rgb, inputs_nir, targets, features_rgb, features_nir):
    full = lambda shape: pl.BlockSpec(shape, lambda i: (0, 0))
    out_rgb, out_nir = pl.pallas_call(
        _fused_loss_kernel,
        grid=(_STEPS,),
        in_specs=[
            pl.BlockSpec(memory_space=pltpu.MemorySpace.SMEM),
            full((_B, _D)),
            full((_B, _D)),
            pl.BlockSpec((_ROWS, _D), lambda i: (i, 0)),
            pl.BlockSpec((_ROWS, _D), lambda i: (i, 0)),
            pl.BlockSpec(memory_space=pl.ANY),
            pl.BlockSpec(memory_space=pl.ANY),
        ],
        out_specs=[full((1, 1)), full((1, 1))],
        out_shape=[jax.ShapeDtypeStruct((1, 1), jnp.float32),
                   jax.ShapeDtypeStruct((1, 1), jnp.float32)],
        scratch_shapes=[
            pltpu.VMEM((_B, _D), jnp.float32),
            pltpu.VMEM((_B, _D), jnp.float32),
            pltpu.VMEM((_B, _D), jnp.float32),
            pltpu.VMEM((_B, _D), jnp.float32),
            pltpu.VMEM((1, _B), jnp.float32),
            pltpu.VMEM((1, _B), jnp.float32),
            pltpu.VMEM((_B, _D), jnp.float32),
            pltpu.VMEM((_B, _D), jnp.float32),
            pltpu.SemaphoreType.DMA,
        ],
        compiler_params=pltpu.CompilerParams(
            dimension_semantics=("arbitrary",)),
    )(targets, inputs_rgb, inputs_nir, features_rgb, features_nir,
      features_rgb, features_nir)
    return (out_rgb.reshape(()), out_nir.reshape(()))
